# baseline (speedup 1.0000x reference)
import functools

import jax
import jax.numpy as jnp
from jax import lax
from jax.experimental import pallas as pl
from jax.experimental.pallas import tpu as pltpu
from jax.experimental.pallas import tpu_sc as plsc

B, P, Q, D = 8, 4096, 4096, 256
PT = 1024
QT = 256
NPT = P // PT
NQT = Q // QT


def _argmin_body(p_ref, q_ref, out_ref, best_val, best_idx):
    b = pl.program_id(0)
    pt = pl.program_id(1)
    qt = pl.program_id(2)

    p = p_ref[0]
    q = q_ref[0]
    pnorm = jnp.sum(p * p, axis=0, keepdims=True)
    qp = jnp.dot(q, p, preferred_element_type=jnp.float32)
    scores = pnorm - 2.0 * qp

    local_min = jnp.min(scores, axis=1)
    col = lax.broadcasted_iota(jnp.int32, (QT, PT), 1)
    local_arg = jnp.min(
        jnp.where(scores == local_min[:, None], col, PT), axis=1
    ) + pt * PT

    @pl.when(pt == 0)
    def _():
        best_val[qt] = local_min
        best_idx[qt] = local_arg

    @pl.when(pt > 0)
    def _():
        bv = best_val[qt]
        bi = best_idx[qt]
        upd = local_min < bv
        best_val[qt] = jnp.where(upd, local_min, bv)
        best_idx[qt] = jnp.where(upd, local_arg, bi)

    out_ref[0, 0] = best_idx[qt] + b * P


def _out_index(b, p, q):
    return (b * NQT + q, 0, 0)


def _nn_indices(prompt_t, query_feats):
    return pl.pallas_call(
        _argmin_body,
        grid=(B, NPT, NQT),
        in_specs=[
            pl.BlockSpec((1, D, PT), lambda b, p, q: (b, 0, p)),
            pl.BlockSpec((1, QT, D), lambda b, p, q: (b, q, 0)),
        ],
        out_specs=pl.BlockSpec((1, 1, QT), _out_index),
        out_shape=jax.ShapeDtypeStruct((B * NQT, 1, QT), jnp.int32),
        scratch_shapes=[
            pltpu.VMEM((NQT, QT), jnp.float32),
            pltpu.VMEM((NQT, QT), jnp.int32),
        ],
        compiler_params=pltpu.CompilerParams(
            dimension_semantics=("arbitrary", "arbitrary", "arbitrary"),
        ),
    )(prompt_t, query_feats)


NC, NS = 2, 16
NW = NC * NS
ROWS = B * Q
ROWS_PER_W = ROWS // NW
CH = 128
NCHUNK = ROWS_PER_W // CH


def _gather_body(table_hbm, idx_hbm, out_hbm, idx_v, rows_v, sem):
    wid = lax.axis_index("s") * NC + lax.axis_index("c")
    for i in range(NCHUNK):
        base = wid * ROWS_PER_W + i * CH
        pltpu.sync_copy(idx_hbm.at[pl.ds(base, CH)], idx_v)
        pltpu.async_copy(table_hbm.at[idx_v], rows_v, sem).wait()
        pltpu.sync_copy(rows_v, out_hbm.at[pl.ds(base, CH)])


@functools.cache
def _sc_gather():
    return pl.kernel(
        _gather_body,
        out_type=jax.ShapeDtypeStruct((ROWS, D), jnp.float32),
        mesh=plsc.VectorSubcoreMesh(core_axis_name="c", subcore_axis_name="s"),
        scratch_types=[
            pltpu.VMEM((CH,), jnp.int32),
            pltpu.VMEM((CH, D), jnp.float32),
            pltpu.SemaphoreType.DMA,
        ],
    )


@jax.jit
def kernel(prompt_feats, query_feats):
    prompt_t = prompt_feats.transpose(0, 2, 1)
    nn_idx = _nn_indices(prompt_t, query_feats)
    idx_flat = nn_idx.reshape(ROWS)
    table = prompt_feats.reshape(ROWS, D)
    out = _sc_gather()(table, idx_flat)
    return out.reshape(B, Q, D)


# speedup vs baseline: 1.8602x; 1.8602x over previous
import functools

import jax
import jax.numpy as jnp
from jax import lax
from jax.experimental import pallas as pl
from jax.experimental.pallas import tpu as pltpu
from jax.experimental.pallas import tpu_sc as plsc

B, P, Q, D = 8, 4096, 4096, 256
QT = 256
NQT = Q // QT
LCH = 128
NCHK = P // LCH


def _argmin_body(p_ref, q_ref, out_ref, pnorm_s):
    b = pl.program_id(0)
    qt = pl.program_id(1)

    p = p_ref[0]

    @pl.when(qt == 0)
    def _():
        pnorm_s[:, :] = jnp.sum(p * p, axis=0, keepdims=True)

    q = q_ref[0]
    q2 = q * -2.0
    qp = jnp.dot(q2, p, preferred_element_type=jnp.float32)
    qnorm = jnp.sum(q * q, axis=1, keepdims=True)
    t1 = qnorm + pnorm_s[:, :]
    scores = t1 + qp

    nodes = []
    for c in range(0, NCHK, 2):
        a = scores[:, c * LCH:(c + 1) * LCH]
        bb = scores[:, (c + 1) * LCH:(c + 2) * LCH]
        t = bb < a
        nodes.append((jnp.where(t, bb, a),
                      jnp.where(t, jnp.int32(c + 1), jnp.int32(c))))
    while len(nodes) > 1:
        nxt = []
        for k in range(0, len(nodes), 2):
            av, ai = nodes[k]
            bv, bi = nodes[k + 1]
            t = bv < av
            nxt.append((jnp.where(t, bv, av), jnp.where(t, bi, ai)))
        nodes = nxt
    m128, c128 = nodes[0]

    lane = lax.broadcasted_iota(jnp.int32, (QT, LCH), 1)
    idx128 = lane * NCHK + c128

    d128 = jnp.sqrt(jnp.maximum(m128, 0.0))
    lmin = jnp.min(d128, axis=1, keepdims=True)
    pick = d128 == lmin
    idx = jnp.min(jnp.where(pick, idx128, P), axis=1)

    out_ref[0, 0] = idx + b * P


def _out_index(b, q):
    return (b * NQT + q, 0, 0)


def _nn_indices(prompt_perm, query_feats):
    return pl.pallas_call(
        _argmin_body,
        grid=(B, NQT),
        in_specs=[
            pl.BlockSpec((1, D, P), lambda b, q: (b, 0, 0)),
            pl.BlockSpec((1, QT, D), lambda b, q: (b, q, 0)),
        ],
        out_specs=pl.BlockSpec((1, 1, QT), _out_index),
        out_shape=jax.ShapeDtypeStruct((B * NQT, 1, QT), jnp.int32),
        scratch_shapes=[
            pltpu.VMEM((1, P), jnp.float32),
        ],
        compiler_params=pltpu.CompilerParams(
            dimension_semantics=("arbitrary", "arbitrary"),
        ),
    )(prompt_perm, query_feats)


NC, NS = 2, 16
NW = NC * NS
ROWS = B * Q
ROWS_PER_W = ROWS // NW
CH = 128
NCHUNK = ROWS_PER_W // CH


def _gather_body(table_hbm, idx_hbm, out_hbm, idx_v, rows_v, sem):
    wid = lax.axis_index("s") * NC + lax.axis_index("c")
    for i in range(NCHUNK):
        base = wid * ROWS_PER_W + i * CH
        pltpu.sync_copy(idx_hbm.at[pl.ds(base, CH)], idx_v)
        pltpu.async_copy(table_hbm.at[idx_v], rows_v, sem).wait()
        pltpu.sync_copy(rows_v, out_hbm.at[pl.ds(base, CH)])


@functools.cache
def _sc_gather():
    return pl.kernel(
        _gather_body,
        out_type=jax.ShapeDtypeStruct((ROWS, D), jnp.float32),
        mesh=plsc.VectorSubcoreMesh(core_axis_name="c", subcore_axis_name="s"),
        scratch_types=[
            pltpu.VMEM((CH,), jnp.int32),
            pltpu.VMEM((CH, D), jnp.float32),
            pltpu.SemaphoreType.DMA,
        ],
    )


@jax.jit
def kernel(prompt_feats, query_feats):
    prompt_t = prompt_feats.transpose(0, 2, 1)
    prompt_perm = (prompt_t.reshape(B, D, LCH, NCHK)
                   .transpose(0, 1, 3, 2).reshape(B, D, P))
    nn_idx = _nn_indices(prompt_perm, query_feats)
    idx_flat = nn_idx.reshape(ROWS)
    table = prompt_feats.reshape(ROWS, D)
    out = _sc_gather()(table, idx_flat)
    return out.reshape(B, Q, D)


# speedup vs baseline: 1.8841x; 1.0128x over previous
import functools

import jax
import jax.numpy as jnp
from jax import lax
from jax.experimental import pallas as pl
from jax.experimental.pallas import tpu as pltpu
from jax.experimental.pallas import tpu_sc as plsc

B, P, Q, D = 8, 4096, 4096, 256
QT = 256
NQT = Q // QT
LCH = 128
NCHK = P // LCH


def _argmin_body(p_ref, q_ref, out_ref, pnorm_s):
    b = pl.program_id(0)
    qt = pl.program_id(1)

    p = p_ref[0]

    @pl.when(qt == 0)
    def _():
        pnorm_s[:, :] = jnp.sum(p * p, axis=0, keepdims=True)

    q = q_ref[0]
    q2 = q * -2.0
    qp = jnp.dot(q2, p, preferred_element_type=jnp.float32)
    qnorm = jnp.sum(q * q, axis=1, keepdims=True)
    t1 = qnorm + pnorm_s[:, :]
    scores = t1 + qp

    nodes = []
    for c in range(0, NCHK, 2):
        a = scores[:, c * LCH:(c + 1) * LCH]
        bb = scores[:, (c + 1) * LCH:(c + 2) * LCH]
        t = bb < a
        nodes.append((jnp.where(t, bb, a),
                      jnp.where(t, jnp.int32(c + 1), jnp.int32(c))))
    while len(nodes) > 1:
        nxt = []
        for k in range(0, len(nodes), 2):
            av, ai = nodes[k]
            bv, bi = nodes[k + 1]
            t = bv < av
            nxt.append((jnp.where(t, bv, av), jnp.where(t, bi, ai)))
        nodes = nxt
    m128, c128 = nodes[0]

    lane = lax.broadcasted_iota(jnp.int32, (QT, LCH), 1)
    idx128 = lane * NCHK + c128

    d128 = jnp.sqrt(jnp.maximum(m128, 0.0))
    lmin = jnp.min(d128, axis=1, keepdims=True)
    pick = d128 == lmin
    idx = jnp.min(jnp.where(pick, idx128, P), axis=1)

    out_ref[0, 0] = idx + b * P


def _out_index(b, q):
    return (b * NQT + q, 0, 0)


def _nn_indices(prompt_perm, query_feats):
    return pl.pallas_call(
        _argmin_body,
        grid=(B, NQT),
        in_specs=[
            pl.BlockSpec((1, D, P), lambda b, q: (b, 0, 0)),
            pl.BlockSpec((1, QT, D), lambda b, q: (b, q, 0)),
        ],
        out_specs=pl.BlockSpec((1, 1, QT), _out_index),
        out_shape=jax.ShapeDtypeStruct((B * NQT, 1, QT), jnp.int32),
        scratch_shapes=[
            pltpu.VMEM((1, P), jnp.float32),
        ],
        compiler_params=pltpu.CompilerParams(
            dimension_semantics=("arbitrary", "arbitrary"),
        ),
    )(prompt_perm, query_feats)


NC, NS = 2, 16
NW = NC * NS
ROWS = B * Q
ROWS_PER_W = ROWS // NW
CH = 128
NCHUNK = ROWS_PER_W // CH


def _gather_chunks(nchunk):
    def body(table_hbm, idx_hbm, out_hbm, idx_v, rows0, rows1,
             g0, g1, s0, s1):
        wid = lax.axis_index("s") * NC + lax.axis_index("c")
        base = wid * nchunk * CH
        pltpu.sync_copy(idx_hbm.at[pl.ds(wid * nchunk, nchunk)], idx_v)
        bufs = (rows0, rows1)
        gsems = (g0, g1)
        ssems = (s0, s1)
        pltpu.async_copy(table_hbm.at[idx_v.at[0]], bufs[0], gsems[0])
        for i in range(nchunk):
            k = i % 2
            kn = (i + 1) % 2
            if i + 1 < nchunk:
                if i >= 1:
                    pltpu.make_async_copy(
                        bufs[kn], out_hbm.at[pl.ds(base + (i - 1) * CH, CH)],
                        ssems[kn]).wait()
                pltpu.async_copy(
                    table_hbm.at[idx_v.at[i + 1]], bufs[kn], gsems[kn])
            pltpu.make_async_copy(
                table_hbm.at[idx_v.at[i]], bufs[k], gsems[k]).wait()
            pltpu.async_copy(
                bufs[k], out_hbm.at[pl.ds(base + i * CH, CH)], ssems[k])
        k_last = (nchunk - 1) % 2
        pltpu.make_async_copy(
            bufs[k_last],
            out_hbm.at[pl.ds(base + (nchunk - 1) * CH, CH)],
            ssems[k_last]).wait()

    return body


@functools.cache
def _sc_gather(rows):
    nchunk = rows // (NW * CH)
    return pl.kernel(
        _gather_chunks(nchunk),
        out_type=jax.ShapeDtypeStruct((rows, D), jnp.float32),
        mesh=plsc.VectorSubcoreMesh(core_axis_name="c", subcore_axis_name="s"),
        scratch_types=[
            pltpu.VMEM((nchunk, CH), jnp.int32),
            pltpu.VMEM((CH, D), jnp.float32),
            pltpu.VMEM((CH, D), jnp.float32),
            pltpu.SemaphoreType.DMA,
            pltpu.SemaphoreType.DMA,
            pltpu.SemaphoreType.DMA,
            pltpu.SemaphoreType.DMA,
        ],
    )


@jax.jit
def kernel(prompt_feats, query_feats):
    prompt_t = prompt_feats.transpose(0, 2, 1)
    prompt_perm = (prompt_t.reshape(B, D, LCH, NCHK)
                   .transpose(0, 1, 3, 2).reshape(B, D, P))
    nn_idx = _nn_indices(prompt_perm, query_feats)
    idx2d = nn_idx.reshape(ROWS // CH, CH)
    table = prompt_feats.reshape(ROWS, D)
    out = _sc_gather(ROWS)(table, idx2d)
    return out.reshape(B, Q, D)
